# Initial kernel scaffold; baseline (speedup 1.0000x reference)
import functools

import jax
import jax.numpy as jnp
from jax import lax
from jax.experimental import pallas as pl
from jax.experimental.pallas import tpu as pltpu
from jax.experimental.pallas import tpu_sc as plsc

N = 10000
D = 128
H = 12
HP = 16
DH = H * D
E_TOT = 320000 + N
NC, NS, LANES = 2, 16, 16
NW = NC * NS
CH1 = 128
CH2 = 64
CPW1 = 81
EPW = CPW1 * CH1
E_PAD = NW * EPW
CPW2 = EPW // CH2
RPT = N // NS
RB = 1000


def _tc_prologue_body(x_ref, we_ref, be_ref, wg_ref, ams_ref, amd_ref,
                      h_ref, xw_ref, as_ref, ad_ref, m_ref):
    i = pl.program_id(0)
    h = jnp.maximum(
        jnp.dot(x_ref[...], we_ref[...], preferred_element_type=jnp.float32)
        + be_ref[...], 0.0)
    xw = jnp.dot(h, wg_ref[...], preferred_element_type=jnp.float32)
    a_s = jnp.dot(xw, ams_ref[...], preferred_element_type=jnp.float32)
    a_d = jnp.dot(xw, amd_ref[...], preferred_element_type=jnp.float32)
    h_ref[...] = h
    xw_ref[...] = xw
    as_ref[...] = a_s
    ad_ref[...] = a_d
    cur = jnp.broadcast_to(jnp.max(a_s, axis=0), (8, HP))

    @pl.when(i == 0)
    def _():
        m_ref[...] = cur

    @pl.when(i > 0)
    def _():
        m_ref[...] = jnp.maximum(m_ref[...], cur)


def _tc_prologue(x, W_emb, b2, W_gat, ams, amd):
    grid = N // RB
    return pl.pallas_call(
        _tc_prologue_body,
        grid=(grid,),
        in_specs=[
            pl.BlockSpec((RB, D), lambda i: (i, 0)),
            pl.BlockSpec((D, D), lambda i: (0, 0)),
            pl.BlockSpec((1, D), lambda i: (0, 0)),
            pl.BlockSpec((D, DH), lambda i: (0, 0)),
            pl.BlockSpec((DH, HP), lambda i: (0, 0)),
            pl.BlockSpec((DH, HP), lambda i: (0, 0)),
        ],
        out_specs=[
            pl.BlockSpec((RB, D), lambda i: (i, 0)),
            pl.BlockSpec((RB, DH), lambda i: (i, 0)),
            pl.BlockSpec((RB, HP), lambda i: (i, 0)),
            pl.BlockSpec((RB, HP), lambda i: (i, 0)),
            pl.BlockSpec((8, HP), lambda i: (0, 0)),
        ],
        out_shape=[
            jax.ShapeDtypeStruct((N, D), jnp.float32),
            jax.ShapeDtypeStruct((N, DH), jnp.float32),
            jax.ShapeDtypeStruct((N, HP), jnp.float32),
            jax.ShapeDtypeStruct((N, HP), jnp.float32),
            jax.ShapeDtypeStruct((8, HP), jnp.float32),
        ],
    )(x, W_emb, b2, W_gat, ams, amd)


def _sc1_body(src_hbm, dst_hbm, as_hbm, ad_hbm, m_hbm,
              p_hbm, den_hbm,
              srcv, dstv, asbuf, adbuf, pbuf, mbuf, zbuf, den_sh):
    cid = lax.axis_index("c")
    sid = lax.axis_index("s")
    wid = sid * NC + cid

    def zrow(j, c):
        zbuf[j, :] = jnp.zeros((LANES,), jnp.float32)
        return c
    lax.fori_loop(0, RPT, zrow, 0)
    pltpu.sync_copy(zbuf, den_sh.at[pl.ds(sid * RPT, RPT)])
    plsc.subcore_barrier()

    pltpu.sync_copy(m_hbm.at[pl.ds(0, 1)], mbuf)
    mv = mbuf[0, :]
    headmask = (lax.iota(jnp.int32, (LANES,)) < H).astype(jnp.float32)
    base_w = wid * EPW

    def chunk(ci, c):
        b0 = base_w + ci * CH1
        pltpu.sync_copy(src_hbm.at[pl.ds(b0, CH1)], srcv)
        pltpu.sync_copy(dst_hbm.at[pl.ds(b0, CH1)], dstv)
        pltpu.sync_copy(as_hbm.at[srcv], asbuf)
        pltpu.sync_copy(ad_hbm.at[dstv], adbuf)

        def edge(i, cc):
            advec = adbuf[i, :]
            a = asbuf[i, :] + advec
            alpha = jnp.where(a >= 0.0, a, 0.2 * a)
            bs = mv + advec
            bnd = jnp.where(bs >= 0.0, bs, 0.2 * bs)
            val = (b0 + i < E_TOT).astype(jnp.float32)
            pbuf[i, :] = jnp.exp(alpha - bnd) * (headmask * val)
            return cc
        lax.fori_loop(0, CH1, edge, 0)
        pltpu.sync_copy(pbuf, den_sh.at[dstv], add=True)
        pltpu.sync_copy(pbuf, p_hbm.at[pl.ds(b0, CH1)])
        return c
    lax.fori_loop(0, CPW1, chunk, 0)
    plsc.subcore_barrier()
    pltpu.sync_copy(den_sh.at[pl.ds(sid * RPT, RPT)],
                    den_hbm.at[cid, pl.ds(sid * RPT, RPT)])


def _sc_pass1(src, dst, a_s, a_d, m8):
    mesh = plsc.VectorSubcoreMesh(core_axis_name="c", subcore_axis_name="s")
    return pl.kernel(
        _sc1_body,
        out_type=(
            jax.ShapeDtypeStruct((E_PAD, HP), jnp.float32),
            jax.ShapeDtypeStruct((NC, N, HP), jnp.float32),
        ),
        mesh=mesh,
        scratch_types=[
            pltpu.VMEM((CH1,), jnp.int32),
            pltpu.VMEM((CH1,), jnp.int32),
            pltpu.VMEM((CH1, HP), jnp.float32),
            pltpu.VMEM((CH1, HP), jnp.float32),
            pltpu.VMEM((CH1, HP), jnp.float32),
            pltpu.VMEM((1, HP), jnp.float32),
            pltpu.VMEM((RPT, HP), jnp.float32),
            pltpu.VMEM_SHARED((N, HP), jnp.float32),
        ],
    )(src, dst, a_s, a_d, m8)


def _tc_invden_body(den_ref, inv_ref):
    inv_ref[...] = 1.0 / (den_ref[0] + den_ref[1] + 1e-16)


def _tc_invden(den):
    grid = N // RB
    return pl.pallas_call(
        _tc_invden_body,
        grid=(grid,),
        in_specs=[pl.BlockSpec((NC, RB, HP), lambda i: (0, i, 0))],
        out_specs=pl.BlockSpec((RB, HP), lambda i: (i, 0)),
        out_shape=jax.ShapeDtypeStruct((N, HP), jnp.float32),
    )(den)


def _sc2_body(src_hbm, dst_hbm, xw_hbm, p_hbm, inv_hbm,
              acc_hbm,
              srcv, dstv, xwbuf, pbuf, invbuf, msgbuf, zbuf, acc_sh):
    cid = lax.axis_index("c")
    sid = lax.axis_index("s")
    wid = sid * NC + cid

    def zrow(j, c):
        for c8 in range(8):
            zbuf[j, pl.ds(c8 * 16, 16)] = jnp.zeros((LANES,), jnp.float32)
        return c
    lax.fori_loop(0, 125, zrow, 0)
    for k in range(5):
        pltpu.sync_copy(zbuf, acc_sh.at[pl.ds(sid * RPT + k * 125, 125)])
    plsc.subcore_barrier()

    base_w = wid * EPW

    def chunk(ci, c):
        b0 = base_w + ci * CH2
        pltpu.sync_copy(src_hbm.at[pl.ds(b0, CH2)], srcv)
        pltpu.sync_copy(dst_hbm.at[pl.ds(b0, CH2)], dstv)
        pltpu.sync_copy(p_hbm.at[pl.ds(b0, CH2)], pbuf)
        pltpu.sync_copy(xw_hbm.at[srcv], xwbuf)
        pltpu.sync_copy(inv_hbm.at[dstv], invbuf)

        def edge(i, cc):
            ws = [pbuf[i, h] * invbuf[i, h] for h in range(H)]
            for c8 in range(8):
                m = ws[0] * xwbuf[i, pl.ds(c8 * 16, 16)]
                for h in range(1, H):
                    m = m + ws[h] * xwbuf[i, pl.ds(h * D + c8 * 16, 16)]
                msgbuf[i, pl.ds(c8 * 16, 16)] = m
            return cc
        lax.fori_loop(0, CH2, edge, 0)
        pltpu.sync_copy(msgbuf, acc_sh.at[dstv], add=True)
        return c
    lax.fori_loop(0, CPW2, chunk, 0)
    plsc.subcore_barrier()
    pltpu.sync_copy(acc_sh.at[pl.ds(sid * RPT, RPT)],
                    acc_hbm.at[cid, pl.ds(sid * RPT, RPT)])


def _sc_pass2(src, dst, xw, p, inv):
    mesh = plsc.VectorSubcoreMesh(core_axis_name="c", subcore_axis_name="s")
    return pl.kernel(
        _sc2_body,
        out_type=jax.ShapeDtypeStruct((NC, N, D), jnp.float32),
        mesh=mesh,
        scratch_types=[
            pltpu.VMEM((CH2,), jnp.int32),
            pltpu.VMEM((CH2,), jnp.int32),
            pltpu.VMEM((CH2, DH), jnp.float32),
            pltpu.VMEM((CH2, HP), jnp.float32),
            pltpu.VMEM((CH2, HP), jnp.float32),
            pltpu.VMEM((CH2, D), jnp.float32),
            pltpu.VMEM((125, D), jnp.float32),
            pltpu.VMEM_SHARED((N, D), jnp.float32),
        ],
    )(src, dst, xw, p, inv)


def _tc_epilogue_body(h_ref, acc_ref, bias_ref, y_ref):
    y = (acc_ref[0] + acc_ref[1]) * (1.0 / H) + bias_ref[...]
    y_ref[...] = jnp.maximum(h_ref[...] + y, 0.0)


def _tc_epilogue(h, acc, bias2):
    grid = N // RB
    return pl.pallas_call(
        _tc_epilogue_body,
        grid=(grid,),
        in_specs=[
            pl.BlockSpec((RB, D), lambda i: (i, 0)),
            pl.BlockSpec((NC, RB, D), lambda i: (0, i, 0)),
            pl.BlockSpec((1, D), lambda i: (0, 0)),
        ],
        out_specs=pl.BlockSpec((RB, D), lambda i: (i, 0)),
        out_shape=jax.ShapeDtypeStruct((N, D), jnp.float32),
    )(h, acc, bias2)


def kernel(x, edge_index, W_emb, b_emb, W_gat, att_src, att_dst, bias_gat):
    loop = jnp.arange(N, dtype=edge_index.dtype)
    src = jnp.concatenate([edge_index[0], loop]).astype(jnp.int32)
    dst = jnp.concatenate([edge_index[1], loop]).astype(jnp.int32)
    pad = E_PAD - E_TOT
    src = jnp.concatenate([src, jnp.zeros((pad,), jnp.int32)])
    dst = jnp.concatenate([dst, jnp.zeros((pad,), jnp.int32)])

    eye = jnp.eye(H, dtype=jnp.float32)
    ams = jnp.pad((att_src[:, :, None] * eye[:, None, :]).reshape(DH, H),
                  ((0, 0), (0, HP - H)))
    amd = jnp.pad((att_dst[:, :, None] * eye[:, None, :]).reshape(DH, H),
                  ((0, 0), (0, HP - H)))
    b2 = b_emb.reshape(1, D)
    bias2 = bias_gat.reshape(1, D)

    h, xw, a_s, a_d, m8 = _tc_prologue(x, W_emb, b2, W_gat, ams, amd)
    p, den = _sc_pass1(src, dst, a_s, a_d, m8)
    inv = _tc_invden(den)
    acc = _sc_pass2(src, dst, xw, p, inv)
    return _tc_epilogue(h, acc, bias2)


# speedup vs baseline: 6.4966x; 6.4966x over previous
import jax
import jax.numpy as jnp
from jax import lax
from jax.experimental import pallas as pl
from jax.experimental.pallas import tpu as pltpu
from jax.experimental.pallas import tpu_sc as plsc

N = 10000
D = 128
H = 12
HP = 16
DH = H * D
E_TOT = 320000 + N
NC, NS, LANES = 2, 16, 16
NW = NC * NS
CH1 = 128
CH2 = 24
CPW1 = 81
EPW = CPW1 * CH1
E_PAD = NW * EPW
CPW2 = EPW // CH2
N_PAD = 10240
RPT = N_PAD // NS
RB = 1000

_SC_PARAMS = pltpu.CompilerParams(use_tc_tiling_on_sc=False)


def _headmask():
    io = lax.iota(jnp.int32, LANES)
    return jnp.where(io < H, 1.0, 0.0)


def _tc_prologue_body(x_ref, we_ref, be_ref, wg_ref, ams_ref, amd_ref,
                      h_ref, xw_ref, as_ref, ad_ref, m_ref):
    i = pl.program_id(0)
    h = jnp.maximum(
        jnp.dot(x_ref[...], we_ref[...], preferred_element_type=jnp.float32)
        + be_ref[...], 0.0)
    xw = jnp.dot(h, wg_ref[...], preferred_element_type=jnp.float32)
    a_s = jnp.dot(xw, ams_ref[...], preferred_element_type=jnp.float32)
    a_d = jnp.dot(xw, amd_ref[...], preferred_element_type=jnp.float32)
    h_ref[...] = h
    xw_ref[...] = xw
    as_ref[...] = a_s
    ad_ref[...] = a_d
    cur = jnp.broadcast_to(jnp.max(a_s, axis=0), (8, HP))

    @pl.when(i == 0)
    def _():
        m_ref[...] = cur

    @pl.when(i > 0)
    def _():
        m_ref[...] = jnp.maximum(m_ref[...], cur)


def _tc_prologue(x, W_emb, b2, W_gat, ams, amd):
    grid = N // RB
    return pl.pallas_call(
        _tc_prologue_body,
        grid=(grid,),
        in_specs=[
            pl.BlockSpec((RB, D), lambda i: (i, 0)),
            pl.BlockSpec((D, D), lambda i: (0, 0)),
            pl.BlockSpec((1, D), lambda i: (0, 0)),
            pl.BlockSpec((D, DH), lambda i: (0, 0)),
            pl.BlockSpec((DH, HP), lambda i: (0, 0)),
            pl.BlockSpec((DH, HP), lambda i: (0, 0)),
        ],
        out_specs=[
            pl.BlockSpec((RB, D), lambda i: (i, 0)),
            pl.BlockSpec((RB, DH), lambda i: (i, 0)),
            pl.BlockSpec((RB, HP), lambda i: (i, 0)),
            pl.BlockSpec((RB, HP), lambda i: (i, 0)),
            pl.BlockSpec((8, HP), lambda i: (0, 0)),
        ],
        out_shape=[
            jax.ShapeDtypeStruct((N, D), jnp.float32),
            jax.ShapeDtypeStruct((N, DH), jnp.float32),
            jax.ShapeDtypeStruct((N, HP), jnp.float32),
            jax.ShapeDtypeStruct((N, HP), jnp.float32),
            jax.ShapeDtypeStruct((8, HP), jnp.float32),
        ],
    )(x, W_emb, b2, W_gat, ams, amd)


def _sc1_body(src_hbm, dst_hbm, as_hbm, ad_hbm, m_hbm,
              p_hbm, den_hbm,
              srcv, dstv, asbuf, adbuf, pbuf, mbuf, zbuf, den_sh):
    cid = lax.axis_index("c")
    sid = lax.axis_index("s")
    wid = sid * NC + cid

    def zrow(j, c):
        zbuf[j, :] = jnp.zeros((LANES,), jnp.float32)
        return c
    lax.fori_loop(0, RPT, zrow, 0)
    pltpu.sync_copy(zbuf, den_sh.at[pl.ds(sid * RPT, RPT)])
    plsc.subcore_barrier()

    pltpu.sync_copy(m_hbm.at[pl.ds(0, 1)], mbuf)
    base_w = wid * EPW

    def chunk(ci, c):
        b0 = base_w + ci * CH1
        pltpu.sync_copy(src_hbm.at[pl.ds(b0, CH1)], srcv)
        pltpu.sync_copy(dst_hbm.at[pl.ds(b0, CH1)], dstv)
        pltpu.sync_copy(as_hbm.at[srcv], asbuf)
        pltpu.sync_copy(ad_hbm.at[dstv], adbuf)

        def edge(i, cc):
            advec = adbuf[i, :]
            a = asbuf[i, :] + advec
            alpha = jnp.where(a >= 0.0, a, 0.2 * a)
            bs = mbuf[0, :] + advec
            bnd = jnp.where(bs >= 0.0, bs, 0.2 * bs)
            pbuf[i, :] = jnp.exp(alpha - bnd) * _headmask()
            return cc
        lax.fori_loop(0, CH1, edge, 0)

        def zpad(i, cc):
            pbuf[i, :] = jnp.zeros((LANES,), jnp.float32)
            return cc
        valid = jnp.clip(E_TOT - b0, 0, CH1)
        lax.fori_loop(valid, CH1, zpad, 0)
        pltpu.sync_copy(pbuf, den_sh.at[dstv], add=True)
        pltpu.sync_copy(pbuf, p_hbm.at[pl.ds(b0, CH1)])
        return c
    lax.fori_loop(0, CPW1, chunk, 0)
    plsc.subcore_barrier()
    pltpu.sync_copy(den_sh.at[pl.ds(sid * RPT, RPT)],
                    den_hbm.at[cid, pl.ds(sid * RPT, RPT)])


def _sc_pass1(src, dst, a_s, a_d, m8):
    mesh = plsc.VectorSubcoreMesh(core_axis_name="c", subcore_axis_name="s")
    return pl.kernel(
        _sc1_body,
        out_type=(
            jax.ShapeDtypeStruct((E_PAD, HP), jnp.float32),
            jax.ShapeDtypeStruct((NC, N_PAD, HP), jnp.float32),
        ),
        mesh=mesh,
        compiler_params=_SC_PARAMS,
        scratch_types=[
            pltpu.VMEM((CH1,), jnp.int32),
            pltpu.VMEM((CH1,), jnp.int32),
            pltpu.VMEM((CH1, HP), jnp.float32),
            pltpu.VMEM((CH1, HP), jnp.float32),
            pltpu.VMEM((CH1, HP), jnp.float32),
            pltpu.VMEM((1, HP), jnp.float32),
            pltpu.VMEM((RPT, HP), jnp.float32),
            pltpu.VMEM_SHARED((N_PAD, HP), jnp.float32),
        ],
    )(src, dst, a_s, a_d, m8)


def _tc_invden_body(den_ref, inv_ref):
    inv_ref[...] = 1.0 / (den_ref[0] + den_ref[1] + 1e-16)


def _tc_invden(den):
    rb = 1280
    return pl.pallas_call(
        _tc_invden_body,
        grid=(N_PAD // rb,),
        in_specs=[pl.BlockSpec((NC, rb, HP), lambda i: (0, i, 0))],
        out_specs=pl.BlockSpec((rb, HP), lambda i: (i, 0)),
        out_shape=jax.ShapeDtypeStruct((N_PAD, HP), jnp.float32),
    )(den)


def _sc2_body(src_hbm, dst_hbm, xw_hbm, p_hbm, inv_hbm,
              acc_hbm,
              srcv, dstv, xwbuf, pbuf, invbuf, msgbuf, zbuf, acc_sh):
    cid = lax.axis_index("c")
    sid = lax.axis_index("s")
    wid = sid * NC + cid

    def zrow(j, c):
        for c8 in range(8):
            zbuf[j, pl.ds(c8 * 16, 16)] = jnp.zeros((LANES,), jnp.float32)
        return c
    lax.fori_loop(0, 32, zrow, 0)
    for k in range(20):
        pltpu.sync_copy(zbuf, acc_sh.at[pl.ds(sid * RPT + k * 32, 32)])
    plsc.subcore_barrier()

    base_w = wid * EPW

    def chunk(ci, c):
        b0 = base_w + ci * CH2
        pltpu.sync_copy(src_hbm.at[pl.ds(b0, CH2)], srcv)
        pltpu.sync_copy(dst_hbm.at[pl.ds(b0, CH2)], dstv)
        pltpu.sync_copy(p_hbm.at[pl.ds(b0, CH2)], pbuf)
        pltpu.sync_copy(xw_hbm.at[srcv], xwbuf)
        pltpu.sync_copy(inv_hbm.at[dstv], invbuf)

        def edge(i, cc):
            w = pbuf[i, :] * invbuf[i, :]
            ws = [w[h] for h in range(H)]
            for c8 in range(8):
                m = ws[0] * xwbuf[i, pl.ds(c8 * 16, 16)]
                for h in range(1, H):
                    m = m + ws[h] * xwbuf[i, pl.ds(h * D + c8 * 16, 16)]
                msgbuf[i, pl.ds(c8 * 16, 16)] = m
            return cc
        lax.fori_loop(0, CH2, edge, 0)
        pltpu.sync_copy(msgbuf, acc_sh.at[dstv], add=True)
        return c
    lax.fori_loop(0, CPW2, chunk, 0)
    plsc.subcore_barrier()
    pltpu.sync_copy(acc_sh.at[pl.ds(sid * RPT, RPT)],
                    acc_hbm.at[cid, pl.ds(sid * RPT, RPT)])


def _sc_pass2(src, dst, xw, p, inv):
    mesh = plsc.VectorSubcoreMesh(core_axis_name="c", subcore_axis_name="s")
    return pl.kernel(
        _sc2_body,
        out_type=jax.ShapeDtypeStruct((NC, N_PAD, D), jnp.float32),
        mesh=mesh,
        compiler_params=_SC_PARAMS,
        scratch_types=[
            pltpu.VMEM((CH2,), jnp.int32),
            pltpu.VMEM((CH2,), jnp.int32),
            pltpu.VMEM((CH2, DH), jnp.float32),
            pltpu.VMEM((CH2, HP), jnp.float32),
            pltpu.VMEM((CH2, HP), jnp.float32),
            pltpu.VMEM((CH2, D), jnp.float32),
            pltpu.VMEM((32, D), jnp.float32),
            pltpu.VMEM_SHARED((N_PAD, D), jnp.float32),
        ],
    )(src, dst, xw, p, inv)


def _tc_epilogue_body(h_ref, acc_ref, bias_ref, y_ref):
    y = (acc_ref[0] + acc_ref[1]) * (1.0 / H) + bias_ref[...]
    y_ref[...] = jnp.maximum(h_ref[...] + y, 0.0)


def _tc_epilogue(h, acc, bias2):
    grid = N // RB
    return pl.pallas_call(
        _tc_epilogue_body,
        grid=(grid,),
        in_specs=[
            pl.BlockSpec((RB, D), lambda i: (i, 0)),
            pl.BlockSpec((NC, RB, D), lambda i: (0, i, 0)),
            pl.BlockSpec((1, D), lambda i: (0, 0)),
        ],
        out_specs=pl.BlockSpec((RB, D), lambda i: (i, 0)),
        out_shape=jax.ShapeDtypeStruct((N, D), jnp.float32),
    )(h, acc, bias2)


def kernel(x, edge_index, W_emb, b_emb, W_gat, att_src, att_dst, bias_gat):
    loop = jnp.arange(N, dtype=edge_index.dtype)
    src = jnp.concatenate([edge_index[0], loop]).astype(jnp.int32)
    dst = jnp.concatenate([edge_index[1], loop]).astype(jnp.int32)
    pad = E_PAD - E_TOT
    src = jnp.concatenate([src, jnp.zeros((pad,), jnp.int32)])
    dst = jnp.concatenate([dst, jnp.zeros((pad,), jnp.int32)])

    eye = jnp.eye(H, dtype=jnp.float32)
    ams = jnp.pad((att_src[:, :, None] * eye[:, None, :]).reshape(DH, H),
                  ((0, 0), (0, HP - H)))
    amd = jnp.pad((att_dst[:, :, None] * eye[:, None, :]).reshape(DH, H),
                  ((0, 0), (0, HP - H)))
    b2 = b_emb.reshape(1, D)
    bias2 = bias_gat.reshape(1, D)

    h, xw, a_s, a_d, m8 = _tc_prologue(x, W_emb, b2, W_gat, ams, amd)
    p, den = _sc_pass1(src, dst, a_s, a_d, m8)
    inv = _tc_invden(den)
    acc = _sc_pass2(src, dst, xw, p, inv)
    return _tc_epilogue(h, acc, bias2)


# speedup vs baseline: 11.7212x; 1.8042x over previous
import jax
import jax.numpy as jnp
from jax import lax
from jax.experimental import pallas as pl
from jax.experimental.pallas import tpu as pltpu
from jax.experimental.pallas import tpu_sc as plsc

N = 10000
D = 128
H = 12
HP = 16
DH = H * D
E_TOT = 320000 + N
NC, NS, LANES = 2, 16, 16
NW = NC * NS
CH1 = 128
CH2 = 24
CPW1 = 81
EPW = CPW1 * CH1
E_PAD = NW * EPW
CPW2 = EPW // CH2
N_PAD = 10240
RPT = N_PAD // NS
RB = 2000
SUP = 8
DW = DH // 2

_SC_PARAMS = pltpu.CompilerParams(use_tc_tiling_on_sc=False)
_SC_PARAMS2 = pltpu.CompilerParams(use_tc_tiling_on_sc=False,
                                   needs_layout_passes=False)


def _headmask():
    io = lax.iota(jnp.int32, LANES)
    return jnp.where(io < H, 1.0, 0.0)


def _tc_prologue_body(x_ref, we_ref, be_ref, wg_ref, ams_ref, amd_ref,
                      h_ref, xw_ref, as_ref, ad_ref, m_ref):
    i = pl.program_id(0)
    h = jnp.maximum(
        jnp.dot(x_ref[...], we_ref[...], preferred_element_type=jnp.float32)
        + be_ref[...], 0.0)
    xw = jnp.dot(h, wg_ref[...], preferred_element_type=jnp.float32)
    a_s = jnp.dot(xw, ams_ref[...], preferred_element_type=jnp.float32)
    a_d = jnp.dot(xw, amd_ref[...], preferred_element_type=jnp.float32)
    h_ref[...] = h
    xw_ref[...] = xw.astype(jnp.bfloat16)
    as_ref[...] = a_s
    ad_ref[...] = a_d
    cur = jnp.broadcast_to(jnp.max(a_s, axis=0), (8, HP))

    @pl.when(i == 0)
    def _():
        m_ref[...] = cur

    @pl.when(i > 0)
    def _():
        m_ref[...] = jnp.maximum(m_ref[...], cur)


def _tc_prologue(x, W_emb, b2, W_gat, ams, amd):
    grid = N // RB
    return pl.pallas_call(
        _tc_prologue_body,
        grid=(grid,),
        in_specs=[
            pl.BlockSpec((RB, D), lambda i: (i, 0)),
            pl.BlockSpec((D, D), lambda i: (0, 0)),
            pl.BlockSpec((1, D), lambda i: (0, 0)),
            pl.BlockSpec((D, DH), lambda i: (0, 0)),
            pl.BlockSpec((DH, HP), lambda i: (0, 0)),
            pl.BlockSpec((DH, HP), lambda i: (0, 0)),
        ],
        out_specs=[
            pl.BlockSpec((RB, D), lambda i: (i, 0)),
            pl.BlockSpec((RB, DH), lambda i: (i, 0)),
            pl.BlockSpec((RB, HP), lambda i: (i, 0)),
            pl.BlockSpec((RB, HP), lambda i: (i, 0)),
            pl.BlockSpec((8, HP), lambda i: (0, 0)),
        ],
        out_shape=[
            jax.ShapeDtypeStruct((N, D), jnp.float32),
            jax.ShapeDtypeStruct((N, DH), jnp.bfloat16),
            jax.ShapeDtypeStruct((N, HP), jnp.float32),
            jax.ShapeDtypeStruct((N, HP), jnp.float32),
            jax.ShapeDtypeStruct((8, HP), jnp.float32),
        ],
    )(x, W_emb, b2, W_gat, ams, amd)


def _sc1_body(src_hbm, dst_hbm, as_hbm, ad_hbm, m_hbm,
              p_hbm, den_hbm,
              srcv, dstv, asbuf, adbuf, pbuf, mbuf, zbuf, den_sh):
    cid = lax.axis_index("c")
    sid = lax.axis_index("s")
    wid = sid * NC + cid

    def zrow(j, c):
        zbuf[j, :] = jnp.zeros((LANES,), jnp.float32)
        return c
    lax.fori_loop(0, RPT, zrow, 0)
    pltpu.sync_copy(zbuf, den_sh.at[pl.ds(sid * RPT, RPT)])
    plsc.subcore_barrier()

    pltpu.sync_copy(m_hbm.at[pl.ds(0, 1)], mbuf)
    base_w = wid * EPW

    def chunk(ci, c):
        b0 = base_w + ci * CH1
        pltpu.sync_copy(src_hbm.at[pl.ds(b0, CH1)], srcv)
        pltpu.sync_copy(dst_hbm.at[pl.ds(b0, CH1)], dstv)
        pltpu.sync_copy(as_hbm.at[srcv], asbuf)
        pltpu.sync_copy(ad_hbm.at[dstv], adbuf)

        def edge(i, cc):
            advec = adbuf[i, :]
            a = asbuf[i, :] + advec
            alpha = jnp.where(a >= 0.0, a, 0.2 * a)
            bs = mbuf[0, :] + advec
            bnd = jnp.where(bs >= 0.0, bs, 0.2 * bs)
            pbuf[i, :] = jnp.exp(alpha - bnd) * _headmask()
            return cc
        lax.fori_loop(0, CH1, edge, 0)

        def zpad(i, cc):
            pbuf[i, :] = jnp.zeros((LANES,), jnp.float32)
            return cc
        valid = jnp.clip(E_TOT - b0, 0, CH1)
        lax.fori_loop(valid, CH1, zpad, 0)
        pltpu.sync_copy(pbuf, den_sh.at[dstv], add=True)
        pltpu.sync_copy(pbuf, p_hbm.at[pl.ds(b0, CH1)])
        return c
    lax.fori_loop(0, CPW1, chunk, 0)
    plsc.subcore_barrier()
    pltpu.sync_copy(den_sh.at[pl.ds(sid * RPT, RPT)],
                    den_hbm.at[cid, pl.ds(sid * RPT, RPT)])


def _sc_pass1(src, dst, a_s, a_d, m8):
    mesh = plsc.VectorSubcoreMesh(core_axis_name="c", subcore_axis_name="s")
    return pl.kernel(
        _sc1_body,
        out_type=(
            jax.ShapeDtypeStruct((E_PAD, HP), jnp.float32),
            jax.ShapeDtypeStruct((NC, N_PAD, HP), jnp.float32),
        ),
        mesh=mesh,
        compiler_params=_SC_PARAMS,
        scratch_types=[
            pltpu.VMEM((CH1,), jnp.int32),
            pltpu.VMEM((CH1,), jnp.int32),
            pltpu.VMEM((CH1, HP), jnp.float32),
            pltpu.VMEM((CH1, HP), jnp.float32),
            pltpu.VMEM((CH1, HP), jnp.float32),
            pltpu.VMEM((1, HP), jnp.float32),
            pltpu.VMEM((RPT, HP), jnp.float32),
            pltpu.VMEM_SHARED((N_PAD, HP), jnp.float32),
        ],
    )(src, dst, a_s, a_d, m8)


def _tc_invden_body(den_ref, inv_ref):
    inv_ref[...] = 1.0 / (den_ref[0] + den_ref[1] + 1e-16)


def _tc_invden(den):
    rb = 1280
    return pl.pallas_call(
        _tc_invden_body,
        grid=(N_PAD // rb,),
        in_specs=[pl.BlockSpec((NC, rb, HP), lambda i: (0, i, 0))],
        out_specs=pl.BlockSpec((rb, HP), lambda i: (i, 0)),
        out_shape=jax.ShapeDtypeStruct((N_PAD, HP), jnp.float32),
    )(den)


def _sc2_body(src_hbm, dst_hbm, xw_hbm, p_hbm, inv_hbm,
              acc_hbm,
              srcall, dstall, xw0, xw1, pb0, pb1, iv0, iv1, mg0, mg1,
              zbuf, acc_sh, gs0, gs1, ms0, ms1):
    cid = lax.axis_index("c")
    sid = lax.axis_index("s")
    wid = sid * NC + cid
    xwb = (xw0, xw1)
    pb = (pb0, pb1)
    ivb = (iv0, iv1)
    mgb = (mg0, mg1)
    gs = (gs0, gs1)
    ms = (ms0, ms1)

    def zrow(j, c):
        for c8 in range(8):
            zbuf[j, pl.ds(c8 * 16, 16)] = jnp.zeros((LANES,), jnp.float32)
        return c
    lax.fori_loop(0, 16, zrow, 0)
    for k in range(RPT // 16):
        pltpu.sync_copy(zbuf, acc_sh.at[pl.ds(sid * RPT + k * 16, 16)])
    plsc.subcore_barrier()

    def issue(ci, b):
        g0 = ci
        pltpu.async_copy(xw_hbm.at[srcall.at[g0]], xwb[b], gs[b])
        pltpu.async_copy(inv_hbm.at[dstall.at[g0]], ivb[b], gs[b])
        return None

    def superchunk(sj, c):
        row0 = wid * CPW2 + sj * SUP
        pltpu.sync_copy(src_hbm.at[pl.ds(row0, SUP)], srcall)
        pltpu.sync_copy(dst_hbm.at[pl.ds(row0, SUP)], dstall)
        for b in range(2):
            issue(b, b)
            pltpu.async_copy(p_hbm.at[pl.ds((row0 + b) * CH2, CH2)],
                             pb[b], gs[b])

        def pair(pj, cc):
            for b in range(2):
                ci = 2 * pj + b
                pltpu.make_async_copy(xw_hbm.at[srcall.at[ci]],
                                      xwb[b], gs[b]).wait()
                pltpu.make_async_copy(inv_hbm.at[dstall.at[ci]],
                                      ivb[b], gs[b]).wait()
                pltpu.make_async_copy(p_hbm.at[pl.ds((row0 + ci) * CH2, CH2)],
                                      pb[b], gs[b]).wait()

                @pl.when(pj > 0)
                def _():
                    pltpu.make_async_copy(mgb[b], acc_sh.at[dstall.at[ci]],
                                          ms[b]).wait()

                def edge(i, ccc):
                    w = pb[b][i, :] * ivb[b][i, :]
                    ws = [w[h] for h in range(H)]
                    for g in range(4):
                        me = None
                        mo = None
                        for h in range(H):
                            vi = xwb[b][i, pl.ds(h * 64 + g * 16, 16)]
                            ev = plsc.bitcast(vi << 16, jnp.float32)
                            od = plsc.bitcast(
                                vi & jnp.int32(-65536), jnp.float32)
                            if me is None:
                                me = ws[h] * ev
                                mo = ws[h] * od
                            else:
                                me = me + ws[h] * ev
                                mo = mo + ws[h] * od
                        mgb[b][i, pl.ds(g * 32, 16)] = me
                        mgb[b][i, pl.ds(g * 32 + 16, 16)] = mo
                    return ccc
                lax.fori_loop(0, CH2, edge, 0)
                pltpu.async_copy(mgb[b], acc_sh.at[dstall.at[ci]], ms[b],
                                 add=True)

                @pl.when(ci + 2 < SUP)
                def _():
                    issue(ci + 2, b)
                    pltpu.async_copy(
                        p_hbm.at[pl.ds((row0 + ci + 2) * CH2, CH2)],
                        pb[b], gs[b])
            return cc
        lax.fori_loop(0, SUP // 2, pair, 0)
        for b in range(2):
            ci = SUP - 2 + b
            pltpu.make_async_copy(mgb[b], acc_sh.at[dstall.at[ci]],
                                  ms[b]).wait()
        return c
    lax.fori_loop(0, CPW2 // SUP, superchunk, 0)
    plsc.subcore_barrier()
    pltpu.sync_copy(acc_sh.at[pl.ds(sid * RPT, RPT)],
                    acc_hbm.at[cid, pl.ds(sid * RPT, RPT)])


def _sc_pass2(src2d, dst2d, xw_i32, p, inv):
    mesh = plsc.VectorSubcoreMesh(core_axis_name="c", subcore_axis_name="s")
    return pl.kernel(
        _sc2_body,
        out_type=jax.ShapeDtypeStruct((NC, N_PAD, D), jnp.float32),
        mesh=mesh,
        compiler_params=_SC_PARAMS2,
        scratch_types=[
            pltpu.VMEM((SUP, CH2), jnp.int32),
            pltpu.VMEM((SUP, CH2), jnp.int32),
            pltpu.VMEM((CH2, DW), jnp.int32),
            pltpu.VMEM((CH2, DW), jnp.int32),
            pltpu.VMEM((CH2, HP), jnp.float32),
            pltpu.VMEM((CH2, HP), jnp.float32),
            pltpu.VMEM((CH2, HP), jnp.float32),
            pltpu.VMEM((CH2, HP), jnp.float32),
            pltpu.VMEM((CH2, D), jnp.float32),
            pltpu.VMEM((CH2, D), jnp.float32),
            pltpu.VMEM((16, D), jnp.float32),
            pltpu.VMEM_SHARED((N_PAD, D), jnp.float32),
            pltpu.SemaphoreType.DMA,
            pltpu.SemaphoreType.DMA,
            pltpu.SemaphoreType.DMA,
            pltpu.SemaphoreType.DMA,
        ],
    )(src2d, dst2d, xw_i32, p, inv)


def _tc_epilogue_body(h_ref, acc_ref, perm_ref, bias_ref, y_ref):
    acc = acc_ref[0] + acc_ref[1]
    y = jnp.dot(acc, perm_ref[...],
                preferred_element_type=jnp.float32) * (1.0 / H) + bias_ref[...]
    y_ref[...] = jnp.maximum(h_ref[...] + y, 0.0)


def _tc_epilogue(h, acc, perm, bias2):
    grid = N // RB
    return pl.pallas_call(
        _tc_epilogue_body,
        grid=(grid,),
        in_specs=[
            pl.BlockSpec((RB, D), lambda i: (i, 0)),
            pl.BlockSpec((NC, RB, D), lambda i: (0, i, 0)),
            pl.BlockSpec((D, D), lambda i: (0, 0)),
            pl.BlockSpec((1, D), lambda i: (0, 0)),
        ],
        out_specs=pl.BlockSpec((RB, D), lambda i: (i, 0)),
        out_shape=jax.ShapeDtypeStruct((N, D), jnp.float32),
    )(h, acc, perm, bias2)


def kernel(x, edge_index, W_emb, b_emb, W_gat, att_src, att_dst, bias_gat):
    loop = jnp.arange(N, dtype=edge_index.dtype)
    src = jnp.concatenate([edge_index[0], loop]).astype(jnp.int32)
    dst = jnp.concatenate([edge_index[1], loop]).astype(jnp.int32)
    pad = E_PAD - E_TOT
    src = jnp.concatenate([src, jnp.zeros((pad,), jnp.int32)])
    dst = jnp.concatenate([dst, jnp.zeros((pad,), jnp.int32)])
    src2d = src.reshape(E_PAD // CH2, CH2)
    dst2d = dst.reshape(E_PAD // CH2, CH2)

    eye = jnp.eye(H, dtype=jnp.float32)
    ams = jnp.pad((att_src[:, :, None] * eye[:, None, :]).reshape(DH, H),
                  ((0, 0), (0, HP - H)))
    amd = jnp.pad((att_dst[:, :, None] * eye[:, None, :]).reshape(DH, H),
                  ((0, 0), (0, HP - H)))
    b2 = b_emb.reshape(1, D)
    bias2 = bias_gat.reshape(1, D)

    grp = jnp.arange(D) // 32
    r = jnp.arange(D) % 32
    true_c = grp * 32 + jnp.where(r < 16, 2 * r, 2 * (r - 16) + 1)
    perm = jax.nn.one_hot(true_c, D, dtype=jnp.float32)

    h, xw_bf, a_s, a_d, m8 = _tc_prologue(x, W_emb, b2, W_gat, ams, amd)
    xw_i32 = jax.lax.bitcast_convert_type(
        xw_bf.reshape(N, DW, 2), jnp.int32)
    p, den = _sc_pass1(src, dst, a_s, a_d, m8)
    inv = _tc_invden(den)
    acc = _sc_pass2(src2d, dst2d, xw_i32, p, inv)
    return _tc_epilogue(h, acc, perm, bias2)
